# Initial kernel scaffold; baseline (speedup 1.0000x reference)
import jax
import jax.numpy as jnp
from jax.experimental import pallas as pl


def kernel(node_feature, edge_index, edge_feature, lens, W_pre, b_pre, W_e, b_e, w1_0, b1_0, w2_0, b2_0, gamma_0, beta_0, w1_1, b1_1, w2_1, b2_1, gamma_1, beta_1):
    raise NotImplementedError("write your pallas kernel here")


# speedup vs baseline: 2.6956x; 2.6956x over previous
import functools

import jax
import jax.numpy as jnp
from jax import lax
from jax.experimental import pallas as pl
from jax.experimental.pallas import tpu as pltpu
from jax.experimental.pallas import tpu_sc as plsc

_N = 10000
_E = 320000
_G = 20
_D = 128
_ED = 16
_H = 128

_NC = 2
_NS = 16
_L = 16
_NW = _NC * _NS
_EPT = _E // _NW
_C = 80
_NCH = _EPT // _C
_RPT = 632
_RPT_LAST = _N - 15 * _RPT
_ZR = 8


def _pre_body(nf_ref, w_ref, b_ref, o_ref):
    o_ref[...] = (
        jnp.dot(nf_ref[...], w_ref[...], preferred_element_type=jnp.float32)
        + b_ref[...]
    )


def _node_pre(node_feature, W_pre, b_pre):
    return pl.pallas_call(
        _pre_body,
        out_shape=jax.ShapeDtypeStruct((_N, _H), jnp.float32),
    )(node_feature, W_pre, b_pre.reshape(1, _H))


def _edge_pre(edge_feature, W_e, b_e):
    BE = 8000
    return pl.pallas_call(
        _pre_body,
        grid=(_E // BE,),
        in_specs=[
            pl.BlockSpec((BE, _ED), lambda i: (i, 0)),
            pl.BlockSpec((_ED, _H), lambda i: (0, 0)),
            pl.BlockSpec((1, _H), lambda i: (0, 0)),
        ],
        out_specs=pl.BlockSpec((BE, _H), lambda i: (i, 0)),
        out_shape=jax.ShapeDtypeStruct((_E, _H), jnp.float32),
    )(edge_feature, W_e, b_e.reshape(1, _H))


def _mlp_bn(h, w1_ref, b1_ref, w2_ref, b2_ref, g_ref, bb_ref):
    h = jnp.dot(h, w1_ref[...], preferred_element_type=jnp.float32) + b1_ref[...]
    h = jnp.where(h > 0, h, 0.01 * h)
    h = jnp.dot(h, w2_ref[...], preferred_element_type=jnp.float32) + b2_ref[...]
    mu = jnp.mean(h, axis=0, keepdims=True)
    var = jnp.mean((h - mu) ** 2, axis=0, keepdims=True)
    return (h - mu) * lax.rsqrt(var + 1e-5) * g_ref[...] + bb_ref[...]


def _mlp_body(x_ref, p0_ref, p1_ref, w1_ref, b1_ref, w2_ref, b2_ref,
              g_ref, bb_ref, o_ref):
    h = x_ref[...] + p0_ref[...] + p1_ref[...]
    o_ref[...] = _mlp_bn(h, w1_ref, b1_ref, w2_ref, b2_ref, g_ref, bb_ref)


def _mlp(x, p0, p1, w1, b1, w2, b2, g, bb):
    return pl.pallas_call(
        _mlp_body,
        out_shape=jax.ShapeDtypeStruct((_N, _H), jnp.float32),
    )(x, p0, p1, w1, b1.reshape(1, _H), w2, b2.reshape(1, _H),
      g.reshape(1, _H), bb.reshape(1, _H))


def _mlp_pool_body(x_ref, p0_ref, p1_ref, w1_ref, b1_ref, w2_ref, b2_ref,
                   g_ref, bb_ref, il_ref, o_ref):
    h = x_ref[...] + p0_ref[...] + p1_ref[...]
    xn = _mlp_bn(h, w1_ref, b1_ref, w2_ref, b2_ref, g_ref, bb_ref)
    col = lax.broadcasted_iota(jnp.int32, (_G, _N), 1)
    row = lax.broadcasted_iota(jnp.int32, (_G, _N), 0)
    sel = jnp.where(col // (_N // _G) == row, 1.0, 0.0).astype(jnp.float32)
    pooled = jnp.dot(sel, xn, preferred_element_type=jnp.float32)
    mean = pooled * il_ref[...]
    nrm = jnp.sqrt(jnp.sum(mean * mean, axis=1, keepdims=True))
    o_ref[...] = mean / jnp.maximum(nrm, 1e-12)


def _mlp_pool(x, p0, p1, w1, b1, w2, b2, g, bb, inv_lens):
    return pl.pallas_call(
        _mlp_pool_body,
        out_shape=jax.ShapeDtypeStruct((_G, _H), jnp.float32),
    )(x, p0, p1, w1, b1.reshape(1, _H), w2, b2.reshape(1, _H),
      g.reshape(1, _H), bb.reshape(1, _H), inv_lens)


def _sc_edge_pass(x, ea, src, dst):
    mesh = plsc.VectorSubcoreMesh(
        core_axis_name="c", subcore_axis_name="s",
        num_cores=_NC, num_subcores=_NS)

    @functools.partial(
        pl.kernel,
        out_type=jax.ShapeDtypeStruct((_NC, _N, _H), jnp.float32),
        mesh=mesh,
        scratch_types=[
            pltpu.VMEM((_C,), jnp.int32),
            pltpu.VMEM((_C,), jnp.int32),
            pltpu.VMEM((_C, _H), jnp.float32),
            pltpu.VMEM((_C, _H), jnp.float32),
            pltpu.VMEM((_ZR, _H), jnp.float32),
            pltpu.VMEM_SHARED((_N, _H), jnp.float32),
            pltpu.SemaphoreType.DMA,
        ],
    )
    def k(x_hbm, ea_hbm, src_hbm, dst_hbm, out_hbm,
          sidx, didx, xrow, eam, zbuf, accum, sem):
        c = lax.axis_index("c")
        s = lax.axis_index("s")
        wid = c * _NS + s

        @pl.loop(0, _ZR)
        def _(r):
            @pl.loop(0, _H, step=_L)
            def _(h0):
                zbuf[r, pl.ds(h0, _L)] = jnp.zeros((_L,), jnp.float32)

        start = s * _RPT

        @pl.loop(0, _RPT // _ZR)
        def _(i):
            @pl.when((s < _NS - 1) | (i < _RPT_LAST // _ZR))
            def _():
                pltpu.sync_copy(zbuf, accum.at[pl.ds(start + i * _ZR, _ZR)])

        plsc.subcore_barrier()

        base_t = wid * _EPT

        @pl.loop(0, _NCH)
        def _(j):
            base = base_t + j * _C
            pltpu.sync_copy(src_hbm.at[pl.ds(base, _C)], sidx)
            pltpu.sync_copy(dst_hbm.at[pl.ds(base, _C)], didx)
            pltpu.async_copy(x_hbm.at[sidx], xrow, sem).wait()
            pltpu.sync_copy(ea_hbm.at[pl.ds(base, _C)], eam)

            @pl.loop(0, _C)
            def _(r):
                @pl.loop(0, _H, step=_L)
                def _(h0):
                    v = xrow[r, pl.ds(h0, _L)] + eam[r, pl.ds(h0, _L)]
                    eam[r, pl.ds(h0, _L)] = jnp.maximum(v, 0.0)

            pltpu.sync_copy(eam, accum.at[didx], add=True)

        plsc.subcore_barrier()

        @pl.when(s < _NS - 1)
        def _():
            pltpu.sync_copy(accum.at[pl.ds(start, _RPT)],
                            out_hbm.at[c, pl.ds(start, _RPT)])

        @pl.when(s == _NS - 1)
        def _():
            pltpu.sync_copy(accum.at[pl.ds(15 * _RPT, _RPT_LAST)],
                            out_hbm.at[c, pl.ds(15 * _RPT, _RPT_LAST)])

    return k(x, ea, src, dst)


def kernel(node_feature, edge_index, edge_feature, lens,
           W_pre, b_pre, W_e, b_e,
           w1_0, b1_0, w2_0, b2_0, gamma_0, beta_0,
           w1_1, b1_1, w2_1, b2_1, gamma_1, beta_1):
    src = edge_index[0]
    dst = edge_index[1]
    x0 = _node_pre(node_feature, W_pre, b_pre)
    ea = _edge_pre(edge_feature, W_e, b_e)
    p = _sc_edge_pass(x0, ea, src, dst)
    x1 = _mlp(x0, p[0], p[1], w1_0, b1_0, w2_0, b2_0, gamma_0, beta_0)
    p2 = _sc_edge_pass(x1, ea, src, dst)
    inv_lens = (1.0 / lens.astype(jnp.float32)).reshape(_G, 1)
    return _mlp_pool(x1, p2[0], p2[1], w1_1, b1_1, w2_1, b2_1,
                     gamma_1, beta_1, inv_lens)


# speedup vs baseline: 4.1759x; 1.5491x over previous
import functools

import jax
import jax.numpy as jnp
from jax import lax
from jax.experimental import pallas as pl
from jax.experimental.pallas import tpu as pltpu
from jax.experimental.pallas import tpu_sc as plsc

_N = 10000
_E = 320000
_G = 20
_D = 128
_ED = 16
_H = 128

_NC = 2
_NS = 16
_L = 16
_NW = _NC * _NS
_EPT = _E // _NW
_C = 40
_NCH = _EPT // _C
_PH = 125
_RPT = 632
_RPT_LAST = _N - 15 * _RPT
_ZR = 8


def _pre_body(nf_ref, w_ref, b_ref, o_ref):
    o_ref[...] = (
        jnp.dot(nf_ref[...], w_ref[...], preferred_element_type=jnp.float32)
        + b_ref[...]
    )


def _node_pre(node_feature, W_pre, b_pre):
    return pl.pallas_call(
        _pre_body,
        out_shape=jax.ShapeDtypeStruct((_N, _H), jnp.float32),
    )(node_feature, W_pre, b_pre.reshape(1, _H))


def _edge_pre(edge_feature, W_e, b_e):
    BE = 8000
    return pl.pallas_call(
        _pre_body,
        grid=(_E // BE,),
        in_specs=[
            pl.BlockSpec((BE, _ED), lambda i: (i, 0)),
            pl.BlockSpec((_ED, _H), lambda i: (0, 0)),
            pl.BlockSpec((1, _H), lambda i: (0, 0)),
        ],
        out_specs=pl.BlockSpec((BE, _H), lambda i: (i, 0)),
        out_shape=jax.ShapeDtypeStruct((_E, _H), jnp.float32),
    )(edge_feature, W_e, b_e.reshape(1, _H))


def _mlp_bn(h, w1_ref, b1_ref, w2_ref, b2_ref, g_ref, bb_ref):
    h = jnp.dot(h, w1_ref[...], preferred_element_type=jnp.float32) + b1_ref[...]
    h = jnp.where(h > 0, h, 0.01 * h)
    h = jnp.dot(h, w2_ref[...], preferred_element_type=jnp.float32) + b2_ref[...]
    mu = jnp.mean(h, axis=0, keepdims=True)
    var = jnp.mean((h - mu) ** 2, axis=0, keepdims=True)
    return (h - mu) * lax.rsqrt(var + 1e-5) * g_ref[...] + bb_ref[...]


def _mlp_body(x_ref, p0_ref, p1_ref, w1_ref, b1_ref, w2_ref, b2_ref,
              g_ref, bb_ref, o_ref):
    h = x_ref[...] + p0_ref[...] + p1_ref[...]
    o_ref[...] = _mlp_bn(h, w1_ref, b1_ref, w2_ref, b2_ref, g_ref, bb_ref)


def _mlp(x, p0, p1, w1, b1, w2, b2, g, bb):
    return pl.pallas_call(
        _mlp_body,
        out_shape=jax.ShapeDtypeStruct((_N, _H), jnp.float32),
    )(x, p0, p1, w1, b1.reshape(1, _H), w2, b2.reshape(1, _H),
      g.reshape(1, _H), bb.reshape(1, _H))


def _mlp_pool_body(x_ref, p0_ref, p1_ref, w1_ref, b1_ref, w2_ref, b2_ref,
                   g_ref, bb_ref, il_ref, o_ref):
    h = x_ref[...] + p0_ref[...] + p1_ref[...]
    xn = _mlp_bn(h, w1_ref, b1_ref, w2_ref, b2_ref, g_ref, bb_ref)
    col = lax.broadcasted_iota(jnp.int32, (_G, _N), 1)
    row = lax.broadcasted_iota(jnp.int32, (_G, _N), 0)
    sel = jnp.where(col // (_N // _G) == row, 1.0, 0.0).astype(jnp.float32)
    pooled = jnp.dot(sel, xn, preferred_element_type=jnp.float32)
    mean = pooled * il_ref[...]
    nrm = jnp.sqrt(jnp.sum(mean * mean, axis=1, keepdims=True))
    o_ref[...] = mean / jnp.maximum(nrm, 1e-12)


def _mlp_pool(x, p0, p1, w1, b1, w2, b2, g, bb, inv_lens):
    return pl.pallas_call(
        _mlp_pool_body,
        out_shape=jax.ShapeDtypeStruct((_G, _H), jnp.float32),
    )(x, p0, p1, w1, b1.reshape(1, _H), w2, b2.reshape(1, _H),
      g.reshape(1, _H), bb.reshape(1, _H), inv_lens)


def _sc_edge_pass(x, ea, packed):
    mesh = plsc.VectorSubcoreMesh(
        core_axis_name="c", subcore_axis_name="s",
        num_cores=_NC, num_subcores=_NS)

    @functools.partial(
        pl.kernel,
        out_type=jax.ShapeDtypeStruct((_NC, _N, _H), jnp.float32),
        mesh=mesh,
        scratch_types=[
            pltpu.VMEM((_PH, _C), jnp.int32),
            pltpu.VMEM((2, _C), jnp.int32),
            pltpu.VMEM((2, _C), jnp.int32),
            pltpu.VMEM((2, _C, _H), jnp.float32),
            pltpu.VMEM((2, _C, _H), jnp.float32),
            pltpu.VMEM_SHARED((_N, _H), jnp.float32),
            pltpu.SemaphoreType.DMA,
            pltpu.SemaphoreType.DMA,
            pltpu.SemaphoreType.DMA,
        ],
    )
    def k(x_hbm, ea_hbm, pk_hbm, out_hbm,
          pki, sidx, didx, xrow, eam, accum, sg, ss0, ss1):
        c = lax.axis_index("c")
        s = lax.axis_index("s")
        wid = c * _NS + s
        ss = (ss0, ss1)

        zsrc = xrow.at[0, pl.ds(0, _ZR)]

        @pl.loop(0, _ZR)
        def _(r):
            @pl.loop(0, _H, step=_L)
            def _(h0):
                xrow[0, r, pl.ds(h0, _L)] = jnp.zeros((_L,), jnp.float32)

        start = s * _RPT

        @pl.loop(0, _RPT // _ZR)
        def _(i):
            @pl.when((s < _NS - 1) | (i < _RPT_LAST // _ZR))
            def _():
                pltpu.sync_copy(zsrc, accum.at[pl.ds(start + i * _ZR, _ZR)])

        plsc.subcore_barrier()

        base_t = wid * _EPT

        def unpack(j, b):
            for off in (0, _L, _C - _L):
                pk = pki[j, pl.ds(off, _L)]
                sidx[b, pl.ds(off, _L)] = pk & 0xFFFF
                didx[b, pl.ds(off, _L)] = lax.shift_right_logical(pk, 16)

        def start_ge(gbase, j, b):
            pltpu.async_copy(x_hbm.at[sidx.at[b]], xrow.at[b], sg)
            pltpu.async_copy(
                ea_hbm.at[pl.ds(base_t + gbase * _C + j * _C, _C)],
                eam.at[b], sg)

        def wait_ge(b):
            pltpu.make_async_copy(x_hbm.at[sidx.at[b]], xrow.at[b],
                                  sg).wait()
            pltpu.make_async_copy(ea_hbm.at[pl.ds(0, _C)], eam.at[b],
                                  sg).wait()

        def start_sc(b):
            pltpu.async_copy(eam.at[b], accum.at[didx.at[b]], ss[b], add=True)

        def wait_sc(b):
            pltpu.make_async_copy(eam.at[b], accum.at[didx.at[b]],
                                  ss[b]).wait()

        def compute(b):
            @pl.loop(0, _C)
            def _(r):
                @pl.loop(0, _H, step=_L)
                def _(h0):
                    v = xrow[b, r, pl.ds(h0, _L)] + eam[b, r, pl.ds(h0, _L)]
                    eam[b, r, pl.ds(h0, _L)] = jnp.maximum(v, 0.0)

        for p in range(_NCH // _PH):
            pltpu.sync_copy(pk_hbm.at[wid, p], pki)
            gb = p * _PH

            unpack(0, 0)
            start_ge(gb, 0, 0)
            wait_ge(0)
            unpack(1, 1)
            start_ge(gb, 1, 1)
            compute(0)
            start_sc(0)

            @pl.loop(1, _PH - 2, step=2)
            def _(j):
                wait_ge(1)
                wait_sc(0)
                unpack(j + 1, 0)
                start_ge(gb, j + 1, 0)
                compute(1)
                start_sc(1)

                wait_ge(0)
                wait_sc(1)
                unpack(j + 2, 1)
                start_ge(gb, j + 2, 1)
                compute(0)
                start_sc(0)

            wait_ge(1)
            wait_sc(0)
            unpack(_PH - 1, 0)
            start_ge(gb, _PH - 1, 0)
            compute(1)
            start_sc(1)

            wait_ge(0)
            wait_sc(1)
            compute(0)
            start_sc(0)
            wait_sc(0)

        plsc.subcore_barrier()

        @pl.when(s < _NS - 1)
        def _():
            pltpu.sync_copy(accum.at[pl.ds(start, _RPT)],
                            out_hbm.at[c, pl.ds(start, _RPT)])

        @pl.when(s == _NS - 1)
        def _():
            pltpu.sync_copy(accum.at[pl.ds(15 * _RPT, _RPT_LAST)],
                            out_hbm.at[c, pl.ds(15 * _RPT, _RPT_LAST)])

    return k(x, ea, packed)


def kernel(node_feature, edge_index, edge_feature, lens,
           W_pre, b_pre, W_e, b_e,
           w1_0, b1_0, w2_0, b2_0, gamma_0, beta_0,
           w1_1, b1_1, w2_1, b2_1, gamma_1, beta_1):
    packed = (edge_index[0] | (edge_index[1] << 16)).reshape(
        _NW, _NCH // _PH, _PH, _C)
    x0 = _node_pre(node_feature, W_pre, b_pre)
    ea = _edge_pre(edge_feature, W_e, b_e)
    p = _sc_edge_pass(x0, ea, packed)
    x1 = _mlp(x0, p[0], p[1], w1_0, b1_0, w2_0, b2_0, gamma_0, beta_0)
    p2 = _sc_edge_pass(x1, ea, packed)
    inv_lens = (1.0 / lens.astype(jnp.float32)).reshape(_G, 1)
    return _mlp_pool(x1, p2[0], p2[1], w1_1, b1_1, w2_1, b2_1,
                     gamma_1, beta_1, inv_lens)


# speedup vs baseline: 5.5812x; 1.3365x over previous
import functools

import jax
import jax.numpy as jnp
from jax import lax
from jax.experimental import pallas as pl
from jax.experimental.pallas import tpu as pltpu
from jax.experimental.pallas import tpu_sc as plsc

_N = 10000
_E = 320000
_G = 20
_D = 128
_ED = 16
_H = 128

_NC = 2
_NS = 16
_L = 16
_NW = _NC * _NS
_EPT = _E // _NW
_C = 40
_NCH = _EPT // _C
_PH = 50
_RPT = 632
_RPT_LAST = _N - 15 * _RPT
_ZR = 8


def _pre_body(nf_ref, w_ref, b_ref, o_ref):
    o_ref[...] = (
        jnp.dot(nf_ref[...], w_ref[...], preferred_element_type=jnp.float32)
        + b_ref[...]
    )


def _node_pre(node_feature, W_pre, b_pre):
    return pl.pallas_call(
        _pre_body,
        out_shape=jax.ShapeDtypeStruct((_N, _H), jnp.float32),
    )(node_feature, W_pre, b_pre.reshape(1, _H))


def _edge_pre(edge_feature, W_e, b_e):
    BE = 8000
    return pl.pallas_call(
        _pre_body,
        grid=(_E // BE,),
        in_specs=[
            pl.BlockSpec((BE, _ED), lambda i: (i, 0)),
            pl.BlockSpec((_ED, _H), lambda i: (0, 0)),
            pl.BlockSpec((1, _H), lambda i: (0, 0)),
        ],
        out_specs=pl.BlockSpec((BE, _H), lambda i: (i, 0)),
        out_shape=jax.ShapeDtypeStruct((_E, _H), jnp.float32),
    )(edge_feature, W_e, b_e.reshape(1, _H))


def _mlp_bn(h, w1_ref, b1_ref, w2_ref, b2_ref, g_ref, bb_ref):
    h = jnp.dot(h, w1_ref[...], preferred_element_type=jnp.float32) + b1_ref[...]
    h = jnp.where(h > 0, h, 0.01 * h)
    h = jnp.dot(h, w2_ref[...], preferred_element_type=jnp.float32) + b2_ref[...]
    mu = jnp.mean(h, axis=0, keepdims=True)
    var = jnp.mean((h - mu) ** 2, axis=0, keepdims=True)
    return (h - mu) * lax.rsqrt(var + 1e-5) * g_ref[...] + bb_ref[...]


def _mlp_body(x_ref, p0_ref, p1_ref, w1_ref, b1_ref, w2_ref, b2_ref,
              g_ref, bb_ref, o_ref):
    h = x_ref[...] + p0_ref[...] + p1_ref[...]
    o_ref[...] = _mlp_bn(h, w1_ref, b1_ref, w2_ref, b2_ref, g_ref, bb_ref)


def _mlp(x, p0, p1, w1, b1, w2, b2, g, bb):
    return pl.pallas_call(
        _mlp_body,
        out_shape=jax.ShapeDtypeStruct((_N, _H), jnp.float32),
    )(x, p0, p1, w1, b1.reshape(1, _H), w2, b2.reshape(1, _H),
      g.reshape(1, _H), bb.reshape(1, _H))


def _mlp_pool_body(x_ref, p0_ref, p1_ref, w1_ref, b1_ref, w2_ref, b2_ref,
                   g_ref, bb_ref, il_ref, o_ref):
    h = x_ref[...] + p0_ref[...] + p1_ref[...]
    xn = _mlp_bn(h, w1_ref, b1_ref, w2_ref, b2_ref, g_ref, bb_ref)
    col = lax.broadcasted_iota(jnp.int32, (_G, _N), 1)
    row = lax.broadcasted_iota(jnp.int32, (_G, _N), 0)
    sel = jnp.where(col // (_N // _G) == row, 1.0, 0.0).astype(jnp.float32)
    pooled = jnp.dot(sel, xn, preferred_element_type=jnp.float32)
    mean = pooled * il_ref[...]
    nrm = jnp.sqrt(jnp.sum(mean * mean, axis=1, keepdims=True))
    o_ref[...] = mean / jnp.maximum(nrm, 1e-12)


def _mlp_pool(x, p0, p1, w1, b1, w2, b2, g, bb, inv_lens):
    return pl.pallas_call(
        _mlp_pool_body,
        out_shape=jax.ShapeDtypeStruct((_G, _H), jnp.float32),
    )(x, p0, p1, w1, b1.reshape(1, _H), w2, b2.reshape(1, _H),
      g.reshape(1, _H), bb.reshape(1, _H), inv_lens)


def _sc_edge_pass(x, ea, packed):
    mesh = plsc.VectorSubcoreMesh(
        core_axis_name="c", subcore_axis_name="s",
        num_cores=_NC, num_subcores=_NS)

    @functools.partial(
        pl.kernel,
        out_type=jax.ShapeDtypeStruct((_NC, _N, _H), jnp.float32),
        mesh=mesh,
        scratch_types=[
            pltpu.VMEM((_PH, _C), jnp.int32),
            pltpu.VMEM((3, _C), jnp.int32),
            pltpu.VMEM((3, _C), jnp.int32),
            pltpu.VMEM((3, _C, _H), jnp.float32),
            pltpu.VMEM((3, _C, _H), jnp.float32),
            pltpu.VMEM_SHARED((_N, _H), jnp.float32),
            pltpu.SemaphoreType.DMA,
            pltpu.SemaphoreType.DMA,
            pltpu.SemaphoreType.DMA,
            pltpu.SemaphoreType.DMA,
        ],
    )
    def k(x_hbm, ea_hbm, pk_hbm, out_hbm,
          pki, sidx, didx, xrow, eam, accum, sg, ss0, ss1, ss2):
        c = lax.axis_index("c")
        s = lax.axis_index("s")
        wid = c * _NS + s
        ss = (ss0, ss1, ss2)

        zsrc = xrow.at[0, pl.ds(0, _ZR)]

        @pl.loop(0, _ZR)
        def _(r):
            @pl.loop(0, _H, step=_L)
            def _(h0):
                xrow[0, r, pl.ds(h0, _L)] = jnp.zeros((_L,), jnp.float32)

        start = s * _RPT

        @pl.loop(0, _RPT // _ZR)
        def _(i):
            @pl.when((s < _NS - 1) | (i < _RPT_LAST // _ZR))
            def _():
                pltpu.sync_copy(zsrc, accum.at[pl.ds(start + i * _ZR, _ZR)])

        plsc.subcore_barrier()

        base_t = wid * _EPT

        def unpack(j, b):
            for off in (0, _L, _C - _L):
                pk = pki[j, pl.ds(off, _L)]
                sidx[b, pl.ds(off, _L)] = pk & 0xFFFF
                didx[b, pl.ds(off, _L)] = lax.shift_right_logical(pk, 16)

        def start_ge(gbase, j, b):
            pltpu.async_copy(x_hbm.at[sidx.at[b]], xrow.at[b], sg)
            pltpu.async_copy(
                ea_hbm.at[pl.ds(base_t + (gbase + j) * _C, _C)],
                eam.at[b], sg)

        def wait_ge(b):
            pltpu.make_async_copy(x_hbm.at[sidx.at[b]], xrow.at[b],
                                  sg).wait()
            pltpu.make_async_copy(ea_hbm.at[pl.ds(0, _C)], eam.at[b],
                                  sg).wait()

        def start_sc(b):
            pltpu.async_copy(eam.at[b], accum.at[didx.at[b]], ss[b], add=True)

        def wait_sc(b):
            pltpu.make_async_copy(eam.at[b], accum.at[didx.at[b]],
                                  ss[b]).wait()

        def compute(b):
            @pl.loop(0, _C)
            def _(r):
                @pl.loop(0, _H, step=_L)
                def _(h0):
                    v = xrow[b, r, pl.ds(h0, _L)] + eam[b, r, pl.ds(h0, _L)]
                    eam[b, r, pl.ds(h0, _L)] = jnp.maximum(v, 0.0)

        def body(gb, j, b, first=False):
            wait_ge(b)
            if not first:
                wait_sc((b + 2) % 3)
            t = (b + 2) % 3

            @pl.when(j + 2 < _PH)
            def _():
                unpack(j + 2, t)
                start_ge(gb, j + 2, t)

            compute(b)
            start_sc(b)

        for p in range(_NCH // _PH):
            pltpu.sync_copy(pk_hbm.at[wid, p], pki)
            gb = p * _PH

            unpack(0, 0)
            start_ge(gb, 0, 0)
            unpack(1, 1)
            start_ge(gb, 1, 1)
            body(gb, 0, 0, first=True)
            body(gb, 1, 1)

            @pl.loop(2, _PH, step=3)
            def _(j):
                body(gb, j, 2)
                body(gb, j + 1, 0)
                body(gb, j + 2, 1)

            wait_sc((_PH - 1) % 3)

        plsc.subcore_barrier()

        @pl.when(s < _NS - 1)
        def _():
            pltpu.sync_copy(accum.at[pl.ds(start, _RPT)],
                            out_hbm.at[c, pl.ds(start, _RPT)])

        @pl.when(s == _NS - 1)
        def _():
            pltpu.sync_copy(accum.at[pl.ds(15 * _RPT, _RPT_LAST)],
                            out_hbm.at[c, pl.ds(15 * _RPT, _RPT_LAST)])

    return k(x, ea, packed)


def kernel(node_feature, edge_index, edge_feature, lens,
           W_pre, b_pre, W_e, b_e,
           w1_0, b1_0, w2_0, b2_0, gamma_0, beta_0,
           w1_1, b1_1, w2_1, b2_1, gamma_1, beta_1):
    packed = (edge_index[0] | (edge_index[1] << 16)).reshape(
        _NW, _NCH // _PH, _PH, _C)
    x0 = _node_pre(node_feature, W_pre, b_pre)
    ea = _edge_pre(edge_feature, W_e, b_e)
    p = _sc_edge_pass(x0, ea, packed)
    x1 = _mlp(x0, p[0], p[1], w1_0, b1_0, w2_0, b2_0, gamma_0, beta_0)
    p2 = _sc_edge_pass(x1, ea, packed)
    inv_lens = (1.0 / lens.astype(jnp.float32)).reshape(_G, 1)
    return _mlp_pool(x1, p2[0], p2[1], w1_1, b1_1, w2_1, b2_1,
                     gamma_1, beta_1, inv_lens)


# speedup vs baseline: 6.6797x; 1.1968x over previous
import functools

import jax
import jax.numpy as jnp
from jax import lax
from jax.experimental import pallas as pl
from jax.experimental.pallas import tpu as pltpu
from jax.experimental.pallas import tpu_sc as plsc

_N = 10000
_E = 320000
_G = 20
_D = 128
_ED = 16
_H = 128

_NC = 2
_NS = 16
_L = 16
_NW = _NC * _NS
_EPT = _E // _NW
_C = 40
_NCH = _EPT // _C
_PH = 50
_RPT = 632
_RPT_LAST = _N - 15 * _RPT
_ZR = 8


def _pre_body(nf_ref, w_ref, b_ref, o_ref):
    o_ref[...] = (
        jnp.dot(nf_ref[...], w_ref[...], preferred_element_type=jnp.float32)
        + b_ref[...]
    )


def _node_pre(node_feature, W_pre, b_pre):
    return pl.pallas_call(
        _pre_body,
        out_shape=jax.ShapeDtypeStruct((_N, _H), jnp.float32),
    )(node_feature, W_pre, b_pre.reshape(1, _H))


def _edge_pre_body(eft_ref, w_ref, b_ref, ei_ref, o_ref, pk_ref):
    o_ref[...] = lax.dot_general(
        eft_ref[...], w_ref[...], (((0,), (0,)), ((), ())),
        preferred_element_type=jnp.float32) + b_ref[...]
    pk_ref[...] = ei_ref[0] | (ei_ref[1] << 16)


def _edge_pre(edge_feature_t, W_e, b_e, edge_index):
    BE = 6400
    return pl.pallas_call(
        _edge_pre_body,
        grid=(_E // BE,),
        in_specs=[
            pl.BlockSpec((_ED, BE), lambda i: (0, i)),
            pl.BlockSpec((_ED, _H), lambda i: (0, 0)),
            pl.BlockSpec((1, _H), lambda i: (0, 0)),
            pl.BlockSpec((2, BE // _C, _C), lambda i: (0, i, 0)),
        ],
        out_specs=[pl.BlockSpec((BE, _H), lambda i: (i, 0)),
                   pl.BlockSpec((BE // _C, _C), lambda i: (i, 0))],
        out_shape=(jax.ShapeDtypeStruct((_E, _H), jnp.float32),
                   jax.ShapeDtypeStruct((_E // _C, _C), jnp.int32)),
    )(edge_feature_t, W_e, b_e.reshape(1, _H),
      edge_index.reshape(2, _E // _C, _C))


def _mlp_bn(h, w1_ref, b1_ref, w2_ref, b2_ref, g_ref, bb_ref):
    h = jnp.dot(h, w1_ref[...], preferred_element_type=jnp.float32) + b1_ref[...]
    h = jnp.where(h > 0, h, 0.01 * h)
    h = jnp.dot(h, w2_ref[...], preferred_element_type=jnp.float32) + b2_ref[...]
    mu = jnp.mean(h, axis=0, keepdims=True)
    var = jnp.mean((h - mu) ** 2, axis=0, keepdims=True)
    return (h - mu) * lax.rsqrt(var + 1e-5) * g_ref[...] + bb_ref[...]


def _mlp_body(x_ref, p0_ref, p1_ref, w1_ref, b1_ref, w2_ref, b2_ref,
              g_ref, bb_ref, o_ref):
    h = x_ref[...] + p0_ref[...] + p1_ref[...]
    o_ref[...] = _mlp_bn(h, w1_ref, b1_ref, w2_ref, b2_ref, g_ref, bb_ref)


def _mlp(x, p0, p1, w1, b1, w2, b2, g, bb):
    return pl.pallas_call(
        _mlp_body,
        out_shape=jax.ShapeDtypeStruct((_N, _H), jnp.float32),
    )(x, p0, p1, w1, b1.reshape(1, _H), w2, b2.reshape(1, _H),
      g.reshape(1, _H), bb.reshape(1, _H))


def _mlp_pool_body(x_ref, p0_ref, p1_ref, w1_ref, b1_ref, w2_ref, b2_ref,
                   g_ref, bb_ref, il_ref, o_ref):
    h = x_ref[...] + p0_ref[...] + p1_ref[...]
    xn = _mlp_bn(h, w1_ref, b1_ref, w2_ref, b2_ref, g_ref, bb_ref)
    col = lax.broadcasted_iota(jnp.int32, (_G, _N), 1)
    row = lax.broadcasted_iota(jnp.int32, (_G, _N), 0)
    sel = jnp.where(col // (_N // _G) == row, 1.0, 0.0).astype(jnp.float32)
    pooled = jnp.dot(sel, xn, preferred_element_type=jnp.float32)
    mean = pooled * il_ref[...]
    nrm = jnp.sqrt(jnp.sum(mean * mean, axis=1, keepdims=True))
    o_ref[...] = mean / jnp.maximum(nrm, 1e-12)


def _mlp_pool(x, p0, p1, w1, b1, w2, b2, g, bb, inv_lens):
    return pl.pallas_call(
        _mlp_pool_body,
        out_shape=jax.ShapeDtypeStruct((_G, _H), jnp.float32),
    )(x, p0, p1, w1, b1.reshape(1, _H), w2, b2.reshape(1, _H),
      g.reshape(1, _H), bb.reshape(1, _H), inv_lens)


def _sc_edge_pass(x, ea, packed):
    mesh = plsc.VectorSubcoreMesh(
        core_axis_name="c", subcore_axis_name="s",
        num_cores=_NC, num_subcores=_NS)

    @functools.partial(
        pl.kernel,
        out_type=jax.ShapeDtypeStruct((_NC, _N, _H), jnp.float32),
        mesh=mesh,
        scratch_types=[
            pltpu.VMEM((_PH, _C), jnp.int32),
            pltpu.VMEM((3, _C), jnp.int32),
            pltpu.VMEM((3, _C), jnp.int32),
            pltpu.VMEM((3, _C, _H), jnp.float32),
            pltpu.VMEM((3, _C, _H), jnp.float32),
            pltpu.VMEM_SHARED((_N, _H), jnp.float32),
            pltpu.SemaphoreType.DMA,
            pltpu.SemaphoreType.DMA,
            pltpu.SemaphoreType.DMA,
            pltpu.SemaphoreType.DMA,
        ],
    )
    def k(x_hbm, ea_hbm, pk_hbm, out_hbm,
          pki, sidx, didx, xrow, eam, accum, sg, ss0, ss1, ss2):
        c = lax.axis_index("c")
        s = lax.axis_index("s")
        wid = c * _NS + s
        ss = (ss0, ss1, ss2)

        zsrc = xrow.at[0, pl.ds(0, _ZR)]

        @pl.loop(0, _ZR)
        def _(r):
            @pl.loop(0, _H, step=_L)
            def _(h0):
                xrow[0, r, pl.ds(h0, _L)] = jnp.zeros((_L,), jnp.float32)

        start = s * _RPT

        @pl.loop(0, _RPT // _ZR)
        def _(i):
            @pl.when((s < _NS - 1) | (i < _RPT_LAST // _ZR))
            def _():
                pltpu.sync_copy(zsrc, accum.at[pl.ds(start + i * _ZR, _ZR)])

        plsc.subcore_barrier()

        base_t = wid * _EPT

        def unpack(j, b):
            for off in (0, _L, _C - _L):
                pk = pki[j, pl.ds(off, _L)]
                sidx[b, pl.ds(off, _L)] = pk & 0xFFFF
                didx[b, pl.ds(off, _L)] = lax.shift_right_logical(pk, 16)

        def start_ge(gbase, j, b):
            pltpu.async_copy(x_hbm.at[sidx.at[b]], xrow.at[b], sg)
            pltpu.async_copy(
                ea_hbm.at[pl.ds(base_t + (gbase + j) * _C, _C)],
                eam.at[b], sg)

        def wait_ge(b):
            pltpu.make_async_copy(x_hbm.at[sidx.at[b]], xrow.at[b],
                                  sg).wait()
            pltpu.make_async_copy(ea_hbm.at[pl.ds(0, _C)], eam.at[b],
                                  sg).wait()

        def start_sc(b):
            pltpu.async_copy(eam.at[b], accum.at[didx.at[b]], ss[b], add=True)

        def wait_sc(b):
            pltpu.make_async_copy(eam.at[b], accum.at[didx.at[b]],
                                  ss[b]).wait()

        def compute(b):
            @pl.loop(0, _C, step=2)
            def _(r):
                for dr in range(2):
                    for h0 in range(0, _H, _L):
                        v = (xrow[b, r + dr, pl.ds(h0, _L)]
                             + eam[b, r + dr, pl.ds(h0, _L)])
                        eam[b, r + dr, pl.ds(h0, _L)] = jnp.maximum(v, 0.0)

        def body(gb, j, b, first=False):
            wait_ge(b)
            if not first:
                wait_sc((b + 2) % 3)
            t = (b + 2) % 3

            @pl.when(j + 2 < _PH)
            def _():
                unpack(j + 2, t)
                start_ge(gb, j + 2, t)

            compute(b)
            start_sc(b)

        for p in range(_NCH // _PH):
            pltpu.sync_copy(pk_hbm.at[wid, p], pki)
            gb = p * _PH

            unpack(0, 0)
            start_ge(gb, 0, 0)
            unpack(1, 1)
            start_ge(gb, 1, 1)
            body(gb, 0, 0, first=True)
            body(gb, 1, 1)

            @pl.loop(2, _PH, step=3)
            def _(j):
                body(gb, j, 2)
                body(gb, j + 1, 0)
                body(gb, j + 2, 1)

            wait_sc((_PH - 1) % 3)

        plsc.subcore_barrier()

        @pl.when(s < _NS - 1)
        def _():
            pltpu.sync_copy(accum.at[pl.ds(start, _RPT)],
                            out_hbm.at[c, pl.ds(start, _RPT)])

        @pl.when(s == _NS - 1)
        def _():
            pltpu.sync_copy(accum.at[pl.ds(15 * _RPT, _RPT_LAST)],
                            out_hbm.at[c, pl.ds(15 * _RPT, _RPT_LAST)])

    return k(x, ea, packed)


def kernel(node_feature, edge_index, edge_feature, lens,
           W_pre, b_pre, W_e, b_e,
           w1_0, b1_0, w2_0, b2_0, gamma_0, beta_0,
           w1_1, b1_1, w2_1, b2_1, gamma_1, beta_1):
    x0 = _node_pre(node_feature, W_pre, b_pre)
    ea, packed = _edge_pre(edge_feature.T, W_e, b_e, edge_index)
    packed = packed.reshape(_NW, _NCH // _PH, _PH, _C)
    p = _sc_edge_pass(x0, ea, packed)
    x1 = _mlp(x0, p[0], p[1], w1_0, b1_0, w2_0, b2_0, gamma_0, beta_0)
    p2 = _sc_edge_pass(x1, ea, packed)
    inv_lens = (1.0 / lens.astype(jnp.float32)).reshape(_G, 1)
    return _mlp_pool(x1, p2[0], p2[1], w1_1, b1_1, w2_1, b2_1,
                     gamma_1, beta_1, inv_lens)
